# baseline (speedup 1.0000x reference)
import functools

import jax
import jax.numpy as jnp
from jax.experimental import pallas as pl
from jax.experimental.pallas import tpu as pltpu

_VOCAB = 1000
_BLK = 256


def _fused_kernel(idx_ref, tgt_ref, table_ref, table_bf_ref,
                  out_ref, loss_ref, lse_ref):
    @pl.when(pl.program_id(0) == 0)
    def _prep():
        tab = table_ref[...]
        m = jnp.max(tab, axis=1, keepdims=True)
        lse_ref[...] = m + jnp.log(
            jnp.sum(jnp.exp(tab - m), axis=1, keepdims=True))
        loss_ref[...] = jnp.zeros((1, 1), jnp.float32)

    idx = idx_ref[0, 0, :]
    tgt = tgt_ref[0, 0, :]
    iota = jax.lax.broadcasted_iota(jnp.int32, (_BLK, _VOCAB), 1)
    onehot = idx[:, None] == iota
    logits = jnp.dot(onehot.astype(jnp.bfloat16), table_bf_ref[...],
                     preferred_element_type=jnp.float32)
    out_ref[...] = logits
    lse_blk = jnp.dot(onehot.astype(jnp.float32), lse_ref[...],
                      preferred_element_type=jnp.float32,
                      precision=jax.lax.Precision.HIGHEST)
    picked = jnp.sum(jnp.where(tgt[:, None] == iota, logits, 0.0), axis=1)
    loss_ref[...] += (jnp.sum(lse_blk) - jnp.sum(picked)).reshape(1, 1)


@functools.partial(jax.jit, static_argnames=())
def kernel(table, index, targets):
    b, t = index.shape
    n = b * t
    nblk = n // _BLK
    idx = index.reshape(nblk, 1, _BLK).astype(jnp.int32)
    tgt = targets.reshape(nblk, 1, _BLK).astype(jnp.int32)
    table_bf = table.astype(jnp.bfloat16)

    logits_flat, loss_sum = pl.pallas_call(
        _fused_kernel,
        grid=(nblk,),
        in_specs=[
            pl.BlockSpec((1, 1, _BLK), lambda i: (i, 0, 0)),
            pl.BlockSpec((1, 1, _BLK), lambda i: (i, 0, 0)),
            pl.BlockSpec((_VOCAB, _VOCAB), lambda i: (0, 0)),
            pl.BlockSpec((_VOCAB, _VOCAB), lambda i: (0, 0)),
        ],
        out_specs=[
            pl.BlockSpec((_BLK, _VOCAB), lambda i: (i, 0)),
            pl.BlockSpec((1, 1), lambda i: (0, 0)),
        ],
        out_shape=[
            jax.ShapeDtypeStruct((n, _VOCAB), jnp.float32),
            jax.ShapeDtypeStruct((1, 1), jnp.float32),
        ],
        scratch_shapes=[pltpu.VMEM((_VOCAB, 1), jnp.float32)],
    )(idx, tgt, table, table_bf)

    logits = logits_flat.reshape(b, t, _VOCAB)
    loss = loss_sum[0, 0] / n
    return (logits, loss)


# speedup vs baseline: 1.2353x; 1.2353x over previous
import functools

import jax
import jax.numpy as jnp
from jax.experimental import pallas as pl
from jax.experimental.pallas import tpu as pltpu

_VOCAB = 1000
_BLK = 256


def _fused_kernel(idx_ref, tgt_ref, table_ref, out_ref, loss_ref, aug_ref):
    @pl.when(pl.program_id(0) == 0)
    def _prep():
        tab = table_ref[...]
        m = jnp.max(tab, axis=1, keepdims=True)
        lse = m + jnp.log(jnp.sum(jnp.exp(tab - m), axis=1, keepdims=True))
        hi = lse.astype(jnp.bfloat16)
        lo = (lse - hi.astype(jnp.float32)).astype(jnp.bfloat16)
        aug_ref[:, :_VOCAB] = tab.astype(jnp.bfloat16)
        aug_ref[:, _VOCAB:_VOCAB + 1] = hi
        aug_ref[:, _VOCAB + 1:_VOCAB + 2] = lo
        loss_ref[...] = jnp.zeros((1, 1), jnp.float32)

    idx = idx_ref[0, 0, :]
    tgt = tgt_ref[0, 0, :]
    iota = jax.lax.broadcasted_iota(jnp.int32, (_BLK, _VOCAB), 1)
    onehot = (idx[:, None] == iota).astype(jnp.bfloat16)
    aug = jnp.dot(onehot, aug_ref[...], preferred_element_type=jnp.float32)
    logits = aug[:, :_VOCAB]
    out_ref[...] = logits
    lse_sum = jnp.sum(aug[:, _VOCAB:_VOCAB + 2])
    picked = jnp.sum(jnp.where(tgt[:, None] == iota, logits, 0.0))
    loss_ref[...] += (lse_sum - picked).reshape(1, 1)


@functools.partial(jax.jit, static_argnames=())
def kernel(table, index, targets):
    b, t = index.shape
    n = b * t
    nblk = n // _BLK
    idx = index.reshape(nblk, 1, _BLK).astype(jnp.int32)
    tgt = targets.reshape(nblk, 1, _BLK).astype(jnp.int32)

    logits_flat, loss_sum = pl.pallas_call(
        _fused_kernel,
        grid=(nblk,),
        in_specs=[
            pl.BlockSpec((1, 1, _BLK), lambda i: (i, 0, 0)),
            pl.BlockSpec((1, 1, _BLK), lambda i: (i, 0, 0)),
            pl.BlockSpec((_VOCAB, _VOCAB), lambda i: (0, 0)),
        ],
        out_specs=[
            pl.BlockSpec((_BLK, _VOCAB), lambda i: (i, 0)),
            pl.BlockSpec((1, 1), lambda i: (0, 0)),
        ],
        out_shape=[
            jax.ShapeDtypeStruct((n, _VOCAB), jnp.float32),
            jax.ShapeDtypeStruct((1, 1), jnp.float32),
        ],
        scratch_shapes=[pltpu.VMEM((_VOCAB, _VOCAB + 2), jnp.bfloat16)],
    )(idx, tgt, table)

    logits = logits_flat.reshape(b, t, _VOCAB)
    loss = loss_sum[0, 0] / n
    return (logits, loss)


# speedup vs baseline: 1.3294x; 1.0761x over previous
import functools

import jax
import jax.numpy as jnp
from jax.experimental import pallas as pl
from jax.experimental.pallas import tpu as pltpu

_VOCAB = 1000
_BLK = 512


def _fused_kernel(idx_ref, tgt_ref, table_ref, out_ref, loss_ref, aug_ref):
    @pl.when(pl.program_id(0) == 0)
    def _prep():
        tab = table_ref[...]
        m = jnp.max(tab, axis=1, keepdims=True)
        lse = m + jnp.log(jnp.sum(jnp.exp(tab - m), axis=1, keepdims=True))
        hi = lse.astype(jnp.bfloat16)
        lo = (lse - hi.astype(jnp.float32)).astype(jnp.bfloat16)
        aug_ref[:, :_VOCAB] = tab.astype(jnp.bfloat16)
        aug_ref[:, _VOCAB:_VOCAB + 1] = hi
        aug_ref[:, _VOCAB + 1:_VOCAB + 2] = lo
        loss_ref[...] = jnp.zeros((1, 1), jnp.float32)

    idx = idx_ref[0, 0, :]
    tgt = tgt_ref[0, 0, :]
    iota = jax.lax.broadcasted_iota(jnp.int32, (_BLK, _VOCAB), 1)
    onehot = (idx[:, None] == iota).astype(jnp.bfloat16)
    aug = jnp.dot(onehot, aug_ref[...], preferred_element_type=jnp.float32)
    logits = aug[:, :_VOCAB]
    out_ref[...] = logits
    lse_sum = jnp.sum(aug[:, _VOCAB:_VOCAB + 2])
    picked = jnp.sum(jnp.where(tgt[:, None] == iota, logits, 0.0))
    loss_ref[...] += (lse_sum - picked).reshape(1, 1)


@functools.partial(jax.jit, static_argnames=())
def kernel(table, index, targets):
    b, t = index.shape
    n = b * t
    nblk = n // _BLK
    idx = index.reshape(nblk, 1, _BLK).astype(jnp.int32)
    tgt = targets.reshape(nblk, 1, _BLK).astype(jnp.int32)

    logits_flat, loss_sum = pl.pallas_call(
        _fused_kernel,
        grid=(nblk,),
        in_specs=[
            pl.BlockSpec((1, 1, _BLK), lambda i: (i, 0, 0)),
            pl.BlockSpec((1, 1, _BLK), lambda i: (i, 0, 0)),
            pl.BlockSpec((_VOCAB, _VOCAB), lambda i: (0, 0)),
        ],
        out_specs=[
            pl.BlockSpec((_BLK, _VOCAB), lambda i: (i, 0)),
            pl.BlockSpec((1, 1), lambda i: (0, 0)),
        ],
        out_shape=[
            jax.ShapeDtypeStruct((n, _VOCAB), jnp.float32),
            jax.ShapeDtypeStruct((1, 1), jnp.float32),
        ],
        scratch_shapes=[pltpu.VMEM((_VOCAB, _VOCAB + 2), jnp.bfloat16)],
    )(idx, tgt, table)

    logits = logits_flat.reshape(b, t, _VOCAB)
    loss = loss_sum[0, 0] / n
    return (logits, loss)


# speedup vs baseline: 1.7184x; 1.2927x over previous
import functools

import jax
import jax.numpy as jnp
from jax.experimental import pallas as pl
from jax.experimental.pallas import tpu as pltpu

_VOCAB = 1000
_BB = 16


def _fused_kernel(idx_ref, tgt_ref, table_ref, out_ref, loss_ref, aug_ref):
    @pl.when(pl.program_id(0) == 0)
    def _prep():
        tab = table_ref[...]
        m = jnp.max(tab, axis=1, keepdims=True)
        lse = m + jnp.log(jnp.sum(jnp.exp(tab - m), axis=1, keepdims=True))
        hi = lse.astype(jnp.bfloat16)
        lo = (lse - hi.astype(jnp.float32)).astype(jnp.bfloat16)
        aug_ref[:, :_VOCAB] = tab.astype(jnp.bfloat16)
        aug_ref[:, _VOCAB:_VOCAB + 1] = hi
        aug_ref[:, _VOCAB + 1:_VOCAB + 2] = lo
        loss_ref[...] = jnp.zeros((1, 1), jnp.float32)

    rows = out_ref.shape[0] * out_ref.shape[1]
    idx = idx_ref[0, 0, :]
    tgt = tgt_ref[0, 0, :]
    iota = jax.lax.broadcasted_iota(jnp.int32, (rows, _VOCAB), 1)
    onehot = (idx[:, None] == iota).astype(jnp.bfloat16)
    res = jnp.dot(onehot, aug_ref[...], preferred_element_type=jnp.float32)
    logits = res[:, :_VOCAB]
    out_ref[...] = logits.reshape(out_ref.shape)
    lse_sum = jnp.sum(res[:, _VOCAB:_VOCAB + 2])
    picked = jnp.sum(jnp.where(tgt[:, None] == iota, logits, 0.0))
    loss_ref[...] += (lse_sum - picked).reshape(1, 1)


@functools.partial(jax.jit, static_argnames=())
def kernel(table, index, targets):
    b, t = index.shape
    n = b * t
    nblk = b // _BB
    rows = _BB * t
    idx = index.reshape(nblk, 1, rows).astype(jnp.int32)
    tgt = targets.reshape(nblk, 1, rows).astype(jnp.int32)

    logits, loss_sum = pl.pallas_call(
        _fused_kernel,
        grid=(nblk,),
        in_specs=[
            pl.BlockSpec((1, 1, rows), lambda i: (i, 0, 0)),
            pl.BlockSpec((1, 1, rows), lambda i: (i, 0, 0)),
            pl.BlockSpec((_VOCAB, _VOCAB), lambda i: (0, 0)),
        ],
        out_specs=[
            pl.BlockSpec((_BB, t, _VOCAB), lambda i: (i, 0, 0)),
            pl.BlockSpec((1, 1), lambda i: (0, 0)),
        ],
        out_shape=[
            jax.ShapeDtypeStruct((b, t, _VOCAB), jnp.float32),
            jax.ShapeDtypeStruct((1, 1), jnp.float32),
        ],
        scratch_shapes=[pltpu.VMEM((_VOCAB, _VOCAB + 2), jnp.bfloat16)],
    )(idx, tgt, table)

    loss = loss_sum[0, 0] / n
    return (logits, loss)
